# Initial kernel scaffold; baseline (speedup 1.0000x reference)
import jax
import jax.numpy as jnp
from jax.experimental import pallas as pl


def kernel(x, emb_table, lin_table, bias, bn0_g, bn0_b, W1, b1, bn1_g, bn1_b, W2, b2, bn2_g, bn2_b, W3, b3):
    raise NotImplementedError("write your pallas kernel here")


# speedup vs baseline: 1.2956x; 1.2956x over previous
import functools

import jax
import jax.numpy as jnp
from jax import lax
from jax.experimental import pallas as pl
from jax.experimental.pallas import tpu as pltpu
from jax.experimental.pallas import tpu_sc as plsc

B = 16384
F = 26
PER = 38461
TOTAL = F * PER
D = 16
H1 = 64
H2 = 32
EPS = 1e-5

NC = 2
NS = 16
NW = NC * NS
RPW = B // NW
CHUNK = 64
NCHUNK = RPW // CHUNK
CH_IDX = CHUNK * F
NSLICE = CH_IDX // 128

_INV = (1.0 + EPS) ** -0.5


def _sc_body(idx_emb, idx_lin, emb_t, lin_t, cross_out, lin_out,
             ei0, ei1, li0, li1, rows0, rows1, lv0, lv1,
             crossbuf, linbuf, sem0, sem1):
    wid = lax.axis_index("s") * NC + lax.axis_index("c")
    ei = (ei0, ei1)
    li = (li0, li1)
    rows = (rows0, rows1)
    lv = (lv0, lv1)
    sems = (sem0, sem1)

    def start(c, slot):
        pltpu.sync_copy(idx_emb.at[wid, c], ei[slot])
        pltpu.sync_copy(idx_lin.at[wid, c], li[slot])
        cps = []
        for j in range(NSLICE):
            cps.append(pltpu.async_copy(
                emb_t.at[ei[slot].at[j]],
                rows[slot].at[pl.ds(j * 128, 128)], sems[slot]))
            cps.append(pltpu.async_copy(
                lin_t.at[li[slot].at[j]],
                lv[slot].at[pl.ds(j * 128, 128)], sems[slot]))
        return cps

    pending = start(0, 0)
    for c in range(NCHUNK):
        slot = c & 1
        nxt = start(c + 1, (c + 1) & 1) if c + 1 < NCHUNK else ()
        for cp in pending:
            cp.wait()
        rbuf = rows[slot]

        def bbody(b, carry, rbuf=rbuf):
            base = b * F
            v = rbuf[base]
            s = v
            q = v * v
            for f in range(1, F):
                v = rbuf[base + f]
                s = s + v
                q = q + v * v
            crossbuf[b] = 0.5 * (s * s - q)
            return carry

        lax.fori_loop(0, CHUNK, bbody, 0)
        pltpu.sync_copy(
            crossbuf, cross_out.at[pl.ds(wid * RPW + c * CHUNK, CHUNK)])

        lvbuf = lv[slot]
        for g in range(CHUNK // 16):
            acc = lvbuf[pl.ds(g * 16, 16)]
            for f in range(1, F):
                acc = acc + lvbuf[pl.ds(f * CHUNK + g * 16, 16)]
            linbuf[pl.ds(c * CHUNK + g * 16, 16)] = acc
        pending = nxt

    pltpu.sync_copy(linbuf, lin_out.at[pl.ds(wid * RPW, RPW)])


_sc_kernel = functools.partial(
    pl.kernel,
    mesh=plsc.VectorSubcoreMesh(core_axis_name="c", subcore_axis_name="s"),
    compiler_params=pltpu.CompilerParams(use_tc_tiling_on_sc=False),
    out_type=(jax.ShapeDtypeStruct((B, D), jnp.float32),
              jax.ShapeDtypeStruct((B,), jnp.float32)),
    scratch_types=[
        pltpu.VMEM((NSLICE, 128), jnp.int32),
        pltpu.VMEM((NSLICE, 128), jnp.int32),
        pltpu.VMEM((NSLICE, 128), jnp.int32),
        pltpu.VMEM((NSLICE, 128), jnp.int32),
        pltpu.VMEM((CH_IDX, D), jnp.float32),
        pltpu.VMEM((CH_IDX, D), jnp.float32),
        pltpu.VMEM((CH_IDX,), jnp.float32),
        pltpu.VMEM((CH_IDX,), jnp.float32),
        pltpu.VMEM((CHUNK, D), jnp.float32),
        pltpu.VMEM((RPW,), jnp.float32),
        pltpu.SemaphoreType.DMA,
        pltpu.SemaphoreType.DMA,
    ],
)(_sc_body)


def _mlp_body(cross_ref, lin_ref, bn0g, bn0b, w1, b1r, bn1g, bn1b,
              w2, b2r, bn2g, bn2b, w3, b3r, biasr, out_ref):
    c = cross_ref[...] * (bn0g[...] * _INV) + bn0b[...]
    h = jnp.dot(c, w1[...], preferred_element_type=jnp.float32) + b1r[...]
    h = jnp.maximum(h * (bn1g[...] * _INV) + bn1b[...], 0.0)
    h = jnp.dot(h, w2[...], preferred_element_type=jnp.float32) + b2r[...]
    h = jnp.maximum(h * (bn2g[...] * _INV) + bn2b[...], 0.0)
    m = jnp.dot(h, w3[...], preferred_element_type=jnp.float32)
    z = m + b3r[...] + biasr[...] + lin_ref[...]
    out_ref[...] = jax.nn.sigmoid(z)


_BLK = 2048


def _full(shape):
    return pl.BlockSpec(shape, lambda i: (0, 0))


_mlp_call = pl.pallas_call(
    _mlp_body,
    grid=(B // _BLK,),
    in_specs=[
        pl.BlockSpec((_BLK, D), lambda i: (i, 0)),
        pl.BlockSpec((_BLK, 1), lambda i: (i, 0)),
        _full((1, D)), _full((1, D)),
        _full((D, H1)), _full((1, H1)), _full((1, H1)), _full((1, H1)),
        _full((H1, H2)), _full((1, H2)), _full((1, H2)), _full((1, H2)),
        _full((H2, 1)), _full((1, 1)), _full((1, 1)),
    ],
    out_specs=pl.BlockSpec((_BLK, 1), lambda i: (i, 0)),
    out_shape=jax.ShapeDtypeStruct((B, 1), jnp.float32),
)


def kernel(x, emb_table, lin_table, bias, bn0_g, bn0_b, W1, b1, bn1_g, bn1_b,
           W2, b2, bn2_g, bn2_b, W3, b3):
    offs = (jnp.arange(F, dtype=jnp.int32) * PER)[None, :]
    idx = x.astype(jnp.int32) + offs
    idx4 = idx.reshape(NW, NCHUNK, CHUNK, F)
    idx_emb = idx4.reshape(NW, NCHUNK, NSLICE, 128)
    idx_lin = jnp.swapaxes(idx4, 2, 3).reshape(NW, NCHUNK, NSLICE, 128)

    cross, lin = _sc_kernel(idx_emb, idx_lin, emb_table,
                            lin_table.reshape(TOTAL))

    out = _mlp_call(
        cross, lin.reshape(B, 1),
        bn0_g.reshape(1, D), bn0_b.reshape(1, D),
        W1, b1.reshape(1, H1), bn1_g.reshape(1, H1), bn1_b.reshape(1, H1),
        W2, b2.reshape(1, H2), bn2_g.reshape(1, H2), bn2_b.reshape(1, H2),
        W3, b3.reshape(1, 1), bias.reshape(1, 1))
    return out.reshape(B)
